# baseline (speedup 1.0000x reference)
import jax
import jax.numpy as jnp
from jax.experimental import pallas as pl
from jax.experimental.pallas import tpu as pltpu

_CH = 512
_NBUF = 3


def _router_body(x_hbm, w_ref, b_ref, o_hbm, xbuf, obuf, insem, outsem):
    n = x_hbm.shape[0]
    nchunks = n // _CH

    def in_copy(i, slot):
        return pltpu.make_async_copy(
            x_hbm.at[pl.ds(i * _CH, _CH), :], xbuf.at[slot], insem.at[slot]
        )

    def out_copy(i, slot):
        return pltpu.make_async_copy(
            obuf.at[slot], o_hbm.at[pl.ds(i * _CH, _CH), :], outsem.at[slot]
        )

    for j in range(_NBUF):
        in_copy(j, j).start()

    def step(i, carry):
        slot = jax.lax.rem(i, _NBUF)
        in_copy(i, slot).wait()
        logits = jnp.dot(
            xbuf[slot], w_ref[...], preferred_element_type=jnp.float32
        )
        logits = logits + b_ref[...]
        m = jnp.max(logits, axis=-1, keepdims=True)
        e = jnp.exp(logits - m)
        p = e * (1.0 / jnp.sum(e, axis=-1, keepdims=True))

        @pl.when(i >= _NBUF)
        def _():
            out_copy(i - _NBUF, slot).wait()

        obuf[slot] = p
        out_copy(i, slot).start()

        @pl.when(i + _NBUF < nchunks)
        def _():
            in_copy(i + _NBUF, slot).start()

        return carry

    jax.lax.fori_loop(0, nchunks, step, 0, unroll=False)

    def drain(j, carry):
        i = nchunks - _NBUF + j
        out_copy(i, jax.lax.rem(i, _NBUF)).wait()
        return carry

    jax.lax.fori_loop(0, _NBUF, drain, 0, unroll=False)


def kernel(x, W, b):
    n, k = x.shape
    ne = W.shape[1]
    b2 = b.reshape(1, ne)
    return pl.pallas_call(
        _router_body,
        in_specs=[
            pl.BlockSpec(memory_space=pltpu.MemorySpace.HBM),
            pl.BlockSpec(memory_space=pltpu.MemorySpace.VMEM),
            pl.BlockSpec(memory_space=pltpu.MemorySpace.VMEM),
        ],
        out_specs=pl.BlockSpec(memory_space=pltpu.MemorySpace.HBM),
        out_shape=jax.ShapeDtypeStruct((n, ne), jnp.float32),
        scratch_shapes=[
            pltpu.VMEM((_NBUF, _CH, k), jnp.float32),
            pltpu.VMEM((_NBUF, _CH, ne), jnp.float32),
            pltpu.SemaphoreType.DMA((_NBUF,)),
            pltpu.SemaphoreType.DMA((_NBUF,)),
        ],
    )(x, W, b2)


# speedup vs baseline: 1.0012x; 1.0012x over previous
import jax
import jax.numpy as jnp
from jax.experimental import pallas as pl
from jax.experimental.pallas import tpu as pltpu

_CH = 256
_NBUF = 6


def _router_body(x_hbm, w_ref, b_ref, o_hbm, xbuf, obuf, insem, outsem):
    n = x_hbm.shape[0]
    nchunks = n // _CH

    def in_copy(i, slot):
        return pltpu.make_async_copy(
            x_hbm.at[pl.ds(i * _CH, _CH), :], xbuf.at[slot], insem.at[slot]
        )

    def out_copy(i, slot):
        return pltpu.make_async_copy(
            obuf.at[slot], o_hbm.at[pl.ds(i * _CH, _CH), :], outsem.at[slot]
        )

    for j in range(_NBUF):
        in_copy(j, j).start()

    def step(i, carry):
        slot = jax.lax.rem(i, _NBUF)
        in_copy(i, slot).wait()
        logits = jnp.dot(
            xbuf[slot], w_ref[...], preferred_element_type=jnp.float32
        )
        logits = logits + b_ref[...]
        m = jnp.max(logits, axis=-1, keepdims=True)
        e = jnp.exp(logits - m)
        p = e * (1.0 / jnp.sum(e, axis=-1, keepdims=True))

        @pl.when(i >= _NBUF)
        def _():
            out_copy(i - _NBUF, slot).wait()

        obuf[slot] = p
        out_copy(i, slot).start()

        @pl.when(i + _NBUF < nchunks)
        def _():
            in_copy(i + _NBUF, slot).start()

        return carry

    jax.lax.fori_loop(0, nchunks, step, 0, unroll=False)

    def drain(j, carry):
        i = nchunks - _NBUF + j
        out_copy(i, jax.lax.rem(i, _NBUF)).wait()
        return carry

    jax.lax.fori_loop(0, _NBUF, drain, 0, unroll=False)


def kernel(x, W, b):
    n, k = x.shape
    ne = W.shape[1]
    b2 = b.reshape(1, ne)
    return pl.pallas_call(
        _router_body,
        in_specs=[
            pl.BlockSpec(memory_space=pltpu.MemorySpace.HBM),
            pl.BlockSpec(memory_space=pltpu.MemorySpace.VMEM),
            pl.BlockSpec(memory_space=pltpu.MemorySpace.VMEM),
        ],
        out_specs=pl.BlockSpec(memory_space=pltpu.MemorySpace.HBM),
        out_shape=jax.ShapeDtypeStruct((n, ne), jnp.float32),
        scratch_shapes=[
            pltpu.VMEM((_NBUF, _CH, k), jnp.float32),
            pltpu.VMEM((_NBUF, _CH, ne), jnp.float32),
            pltpu.SemaphoreType.DMA((_NBUF,)),
            pltpu.SemaphoreType.DMA((_NBUF,)),
        ],
    )(x, W, b2)


# speedup vs baseline: 1.0102x; 1.0090x over previous
import jax
import jax.numpy as jnp
from jax.experimental import pallas as pl
from jax.experimental.pallas import tpu as pltpu

_CH = 256
_NBUF = 4


def _router_body(x_hbm, w_ref, b_ref, o_hbm, xbuf, obuf, insem, outsem):
    n = x_hbm.shape[0]
    nchunks = n // _CH

    def in_copy(i, slot):
        return pltpu.make_async_copy(
            x_hbm.at[pl.ds(i * _CH, _CH), :], xbuf.at[slot], insem.at[slot]
        )

    def out_copy(i, slot):
        return pltpu.make_async_copy(
            obuf.at[slot], o_hbm.at[pl.ds(i * _CH, _CH), :], outsem.at[slot]
        )

    for j in range(_NBUF):
        in_copy(j, j).start()

    def step(i, carry):
        slot = jax.lax.rem(i, _NBUF)
        in_copy(i, slot).wait()
        logits = jnp.dot(
            xbuf[slot], w_ref[...], preferred_element_type=jnp.float32
        )
        logits = logits + b_ref[...].reshape(1, -1)
        m = jnp.max(logits, axis=-1, keepdims=True)
        e = jnp.exp(logits - m)
        p = e * (1.0 / jnp.sum(e, axis=-1, keepdims=True))

        @pl.when(i >= _NBUF)
        def _():
            out_copy(i - _NBUF, slot).wait()

        obuf[slot] = p
        out_copy(i, slot).start()

        @pl.when(i + _NBUF < nchunks)
        def _():
            in_copy(i + _NBUF, slot).start()

        return carry

    jax.lax.fori_loop(0, nchunks, step, 0, unroll=False)

    def drain(j, carry):
        i = nchunks - _NBUF + j
        out_copy(i, jax.lax.rem(i, _NBUF)).wait()
        return carry

    jax.lax.fori_loop(0, _NBUF, drain, 0, unroll=False)


def kernel(x, W, b):
    n, k = x.shape
    ne = W.shape[1]
    return pl.pallas_call(
        _router_body,
        in_specs=[
            pl.BlockSpec(memory_space=pltpu.MemorySpace.HBM),
            pl.BlockSpec(memory_space=pltpu.MemorySpace.VMEM),
            pl.BlockSpec(memory_space=pltpu.MemorySpace.VMEM),
        ],
        out_specs=pl.BlockSpec(memory_space=pltpu.MemorySpace.HBM),
        out_shape=jax.ShapeDtypeStruct((n, ne), jnp.float32),
        scratch_shapes=[
            pltpu.VMEM((_NBUF, _CH, k), jnp.float32),
            pltpu.VMEM((_NBUF, _CH, ne), jnp.float32),
            pltpu.SemaphoreType.DMA((_NBUF,)),
            pltpu.SemaphoreType.DMA((_NBUF,)),
        ],
    )(x, W, b)


# speedup vs baseline: 1.0144x; 1.0042x over previous
import jax
import jax.numpy as jnp
from jax.experimental import pallas as pl
from jax.experimental.pallas import tpu as pltpu

_CH = 256
_NBUF = 4


def _router_body(x_hbm, w_ref, b_ref, o_ref, xbuf, insem):
    n = x_hbm.shape[0]
    nchunks = n // _CH

    def in_copy(i, slot):
        return pltpu.make_async_copy(
            x_hbm.at[pl.ds(i * _CH, _CH), :], xbuf.at[slot], insem.at[slot]
        )

    for j in range(_NBUF):
        in_copy(j, j).start()

    def step(i, carry):
        slot = jax.lax.rem(i, _NBUF)
        in_copy(i, slot).wait()
        logits = jnp.dot(
            xbuf[slot], w_ref[...], preferred_element_type=jnp.float32
        )
        logits = logits + b_ref[...].reshape(1, -1)
        m = jnp.max(logits, axis=-1, keepdims=True)
        e = jnp.exp(logits - m)
        o_ref[pl.ds(i * _CH, _CH), :] = e * (
            1.0 / jnp.sum(e, axis=-1, keepdims=True)
        )

        @pl.when(i + _NBUF < nchunks)
        def _():
            in_copy(i + _NBUF, slot).start()

        return carry

    jax.lax.fori_loop(0, nchunks, step, 0, unroll=False)


def kernel(x, W, b):
    n, k = x.shape
    ne = W.shape[1]
    return pl.pallas_call(
        _router_body,
        in_specs=[
            pl.BlockSpec(memory_space=pltpu.MemorySpace.HBM),
            pl.BlockSpec(memory_space=pltpu.MemorySpace.VMEM),
            pl.BlockSpec(memory_space=pltpu.MemorySpace.VMEM),
        ],
        out_specs=pl.BlockSpec(memory_space=pltpu.MemorySpace.VMEM),
        out_shape=jax.ShapeDtypeStruct((n, ne), jnp.float32),
        scratch_shapes=[
            pltpu.VMEM((_NBUF, _CH, k), jnp.float32),
            pltpu.SemaphoreType.DMA((_NBUF,)),
        ],
    )(x, W, b)
